# Initial kernel scaffold; baseline (speedup 1.0000x reference)
import jax
import jax.numpy as jnp
from jax.experimental import pallas as pl


def kernel(x, edge_index):
    raise NotImplementedError("write your pallas kernel here")


# speedup vs baseline: 11.0314x; 11.0314x over previous
import functools

import jax
import jax.numpy as jnp
from jax import lax
from jax.experimental import pallas as pl
from jax.experimental.pallas import tpu as pltpu
from jax.experimental.pallas import tpu_sc as plsc

N_NODES = 10000
DIM = 128
HALF = DIM // 2
K_ITERS = 10

N_TILES = 16
NPAD = 10240
RPT = NPAD // N_TILES
RBLK = 128
N_RBLK = RPT // RBLK
CH = 128
IB = 8
NLANE = 16


def _fast_rsqrt(d):
    half = 0.5 * d
    i = lax.bitcast_convert_type(d, jnp.int32)
    y = lax.bitcast_convert_type(
        jnp.int32(0x5F3759DF) - lax.shift_right_arithmetic(i, 1), jnp.float32
    )
    for _ in range(3):
        y = y * (1.5 - half * y * y)
    return y


def _make_appnp(n_iblk):
    mesh = plsc.VectorSubcoreMesh(core_axis_name="c", subcore_axis_name="s")

    @functools.partial(
        pl.kernel,
        out_type=jax.ShapeDtypeStruct((2, NPAD, HALF), jnp.float32),
        mesh=mesh,
        compiler_params=pltpu.CompilerParams(use_tc_tiling_on_sc=False),
        scratch_types=[
            pltpu.VMEM_SHARED((NPAD, HALF), jnp.float32),
            pltpu.VMEM_SHARED((NPAD, HALF), jnp.float32),
            pltpu.VMEM((IB, CH), jnp.int32),
            pltpu.VMEM((IB, CH), jnp.int32),
            pltpu.VMEM((CH, HALF), jnp.float32),
            pltpu.VMEM((CH, HALF), jnp.float32),
            pltpu.VMEM((RBLK, HALF), jnp.float32),
            pltpu.VMEM((RPT, NLANE), jnp.float32),
        ],
    )
    def appnp(xs, srci_h, dsti_h, out_h, z_sh, s_sh, sbuf, dbuf, rb0, rb1,
              bbuf, asl):
        c = lax.axis_index("c")
        s = lax.axis_index("s")
        row0 = s * RPT

        zero16 = jnp.zeros((NLANE,), jnp.float32)
        one16 = jnp.ones((NLANE,), jnp.float32)

        @pl.loop(0, CH)
        def _(i):
            for j in range(HALF // NLANE):
                rb0[i, pl.ds(j * NLANE, NLANE)] = zero16
                rb1[i, pl.ds(j * NLANE, NLANE)] = one16

        for blk in range(N_RBLK):
            pltpu.sync_copy(rb0, s_sh.at[pl.ds(row0 + blk * RBLK, RBLK)])
        plsc.subcore_barrier()

        @pl.loop(0, n_iblk)
        def _(ib):
            pltpu.sync_copy(dsti_h.at[s].at[ib], dbuf)
            for ct in range(IB):
                pltpu.sync_copy(rb1, s_sh.at[dbuf.at[ct]], add=True)

        plsc.subcore_barrier()

        for blk in range(N_RBLK):
            r0 = row0 + blk * RBLK
            pltpu.sync_copy(s_sh.at[pl.ds(r0, RBLK)], rb0)
            pltpu.sync_copy(xs.at[c].at[pl.ds(r0, RBLK)], bbuf)

            @pl.loop(0, RBLK)
            def _(v):
                vg = blk * RBLK + v
                deg = rb0[v, pl.ds(0, NLANE)] + 1.0
                dsv = _fast_rsqrt(deg)
                asl[vg, pl.ds(0, NLANE)] = 0.9 * dsv * dsv
                for j in range(HALF // NLANE):
                    b = 0.1 * dsv * bbuf[v, pl.ds(j * NLANE, NLANE)]
                    bbuf[v, pl.ds(j * NLANE, NLANE)] = b
                    rb0[v, pl.ds(j * NLANE, NLANE)] = 10.0 * b

            pltpu.sync_copy(bbuf, out_h.at[c].at[pl.ds(r0, RBLK)])
            pltpu.sync_copy(rb0, z_sh.at[pl.ds(r0, RBLK)])
            pltpu.sync_copy(rb0, s_sh.at[pl.ds(r0, RBLK)])

        plsc.subcore_barrier()

        def edge_pass():
            @pl.loop(0, n_iblk)
            def _(ib):
                pltpu.sync_copy(srci_h.at[s].at[ib], sbuf)
                pltpu.sync_copy(dsti_h.at[s].at[ib], dbuf)
                for ct in range(IB):
                    pltpu.sync_copy(z_sh.at[sbuf.at[ct]], rb0)
                    pltpu.sync_copy(rb0, s_sh.at[dbuf.at[ct]], add=True)

        @pl.loop(0, K_ITERS - 1)
        def _(k):
            edge_pass()
            plsc.subcore_barrier()

            for blk in range(N_RBLK):
                r0 = row0 + blk * RBLK
                pltpu.sync_copy(s_sh.at[pl.ds(r0, RBLK)], rb0)
                pltpu.sync_copy(out_h.at[c].at[pl.ds(r0, RBLK)], bbuf)

                @pl.loop(0, RBLK)
                def _(v):
                    vg = blk * RBLK + v
                    av = asl[vg, pl.ds(0, NLANE)]
                    for j in range(HALF // NLANE):
                        sj = rb0[v, pl.ds(j * NLANE, NLANE)]
                        rb0[v, pl.ds(j * NLANE, NLANE)] = (
                            av * sj + bbuf[v, pl.ds(j * NLANE, NLANE)]
                        )

                pltpu.sync_copy(rb0, z_sh.at[pl.ds(r0, RBLK)])
                pltpu.sync_copy(rb0, s_sh.at[pl.ds(r0, RBLK)])

            plsc.subcore_barrier()

        edge_pass()
        plsc.subcore_barrier()

        for blk in range(N_RBLK):
            r0 = row0 + blk * RBLK
            pltpu.sync_copy(s_sh.at[pl.ds(r0, RBLK)], rb0)
            pltpu.sync_copy(out_h.at[c].at[pl.ds(r0, RBLK)], bbuf)

            @pl.loop(0, RBLK)
            def _(v):
                vg = blk * RBLK + v
                av = asl[vg, pl.ds(0, NLANE)]
                rv = _fast_rsqrt(av * (1.0 / 0.9))
                for j in range(HALF // NLANE):
                    sj = rb0[v, pl.ds(j * NLANE, NLANE)]
                    zj = av * sj + bbuf[v, pl.ds(j * NLANE, NLANE)]
                    rb0[v, pl.ds(j * NLANE, NLANE)] = jnp.maximum(zj, 0.0) * rv

            pltpu.sync_copy(rb0, out_h.at[c].at[pl.ds(r0, RBLK)])

    return appnp


def kernel(x, edge_index):
    n, d = x.shape
    e = edge_index.shape[1]
    assert n == N_NODES and d == DIM

    unit = N_TILES * CH * IB
    n_iblk = -(-e // unit)
    epad = n_iblk * unit

    pad = N_NODES + (jnp.arange(epad - e, dtype=jnp.int32) % (NPAD - n))
    src = jnp.concatenate([edge_index[0], pad]).reshape(N_TILES, n_iblk, IB, CH)
    dst = jnp.concatenate([edge_index[1], pad]).reshape(N_TILES, n_iblk, IB, CH)

    xp = jnp.pad(x, ((0, NPAD - n), (0, 0)))
    xs = jnp.stack([xp[:, :HALF], xp[:, HALF:]])

    out = _make_appnp(n_iblk)(xs, src, dst)
    return jnp.concatenate([out[0, :n], out[1, :n]], axis=1)


# speedup vs baseline: 16.4269x; 1.4891x over previous
import functools

import jax
import jax.numpy as jnp
from jax import lax
from jax.experimental import pallas as pl
from jax.experimental.pallas import tpu as pltpu
from jax.experimental.pallas import tpu_sc as plsc

N_NODES = 10000
DIM = 128
HALF = DIM // 2
K_ITERS = 10

N_TILES = 16
NPAD = 10240
RPT = NPAD // N_TILES
RBLK = 128
N_RBLK = RPT // RBLK
CH = 128
IB = 8
NLANE = 16


def _fast_rsqrt(d):
    half = 0.5 * d
    i = lax.bitcast_convert_type(d, jnp.int32)
    y = lax.bitcast_convert_type(
        jnp.int32(0x5F3759DF) - lax.shift_right_arithmetic(i, 1), jnp.float32
    )
    for _ in range(3):
        y = y * (1.5 - half * y * y)
    return y


def _make_appnp(n_iblk):
    mesh = plsc.VectorSubcoreMesh(core_axis_name="c", subcore_axis_name="s")

    @functools.partial(
        pl.kernel,
        out_type=jax.ShapeDtypeStruct((2, NPAD, HALF), jnp.float32),
        mesh=mesh,
        compiler_params=pltpu.CompilerParams(use_tc_tiling_on_sc=False),
        scratch_types=[
            pltpu.VMEM_SHARED((NPAD, HALF), jnp.float32),
            pltpu.VMEM_SHARED((NPAD, HALF), jnp.float32),
            pltpu.VMEM((IB, CH), jnp.int32),
            pltpu.VMEM((IB, CH), jnp.int32),
            pltpu.VMEM((IB, CH), jnp.int32),
            pltpu.VMEM((IB, CH), jnp.int32),
            pltpu.VMEM((CH, HALF), jnp.float32),
            pltpu.VMEM((CH, HALF), jnp.float32),
            pltpu.VMEM((RBLK, HALF), jnp.float32),
            pltpu.VMEM((RPT, NLANE), jnp.float32),
            pltpu.SemaphoreType.DMA,
            pltpu.SemaphoreType.DMA,
            pltpu.SemaphoreType.DMA,
            pltpu.SemaphoreType.DMA,
            pltpu.SemaphoreType.DMA,
        ],
    )
    def appnp(xs, srci_h, dsti_h, out_h, z_sh, s_sh, sb0, sb1, db0, db1,
              rb0, rb1, bbuf, asl, gsem0, gsem1, isems, isemd, ssem):
        c = lax.axis_index("c")
        s = lax.axis_index("s")
        row0 = s * RPT

        sbufs = (sb0, sb1)
        dbufs = (db0, db1)
        rbufs = (rb0, rb1)
        gsems = (gsem0, gsem1)

        zero16 = jnp.zeros((NLANE,), jnp.float32)
        one16 = jnp.ones((NLANE,), jnp.float32)

        @pl.loop(0, CH)
        def _(i):
            for j in range(HALF // NLANE):
                rb0[i, pl.ds(j * NLANE, NLANE)] = zero16
                rb1[i, pl.ds(j * NLANE, NLANE)] = one16

        for blk in range(N_RBLK):
            pltpu.sync_copy(rb0, s_sh.at[pl.ds(row0 + blk * RBLK, RBLK)])
        plsc.subcore_barrier()

        pltpu.sync_copy(dsti_h.at[s].at[0], db0)
        pltpu.async_copy(dsti_h.at[s].at[1], db1, isemd)

        @pl.loop(0, n_iblk // 2)
        def _(ibp):
            for par in range(2):
                ib = ibp * 2 + par
                cd = dbufs[par]
                nd = dbufs[1 - par]
                for ct in range(IB):
                    pltpu.async_copy(rb1, s_sh.at[cd.at[ct]], ssem, add=True)
                for ct in range(IB):
                    pltpu.make_async_copy(
                        rb1, s_sh.at[cd.at[ct]], ssem).wait()
                pltpu.make_async_copy(dsti_h.at[s].at[ib + 1], nd, isemd).wait()
                pltpu.async_copy(dsti_h.at[s].at[ib + 2], cd, isemd)

        pltpu.make_async_copy(dsti_h.at[s].at[n_iblk + 1], db1, isemd).wait()
        plsc.subcore_barrier()

        for blk in range(N_RBLK):
            r0 = row0 + blk * RBLK
            pltpu.sync_copy(s_sh.at[pl.ds(r0, RBLK)], rb0)
            pltpu.sync_copy(xs.at[c].at[pl.ds(r0, RBLK)], bbuf)

            @pl.loop(0, RBLK)
            def _(v):
                vg = blk * RBLK + v
                deg = rb0[v, pl.ds(0, NLANE)] + 1.0
                dsv = _fast_rsqrt(deg)
                asl[vg, pl.ds(0, NLANE)] = 0.9 * dsv * dsv
                for j in range(HALF // NLANE):
                    b = 0.1 * dsv * bbuf[v, pl.ds(j * NLANE, NLANE)]
                    bbuf[v, pl.ds(j * NLANE, NLANE)] = b
                    rb0[v, pl.ds(j * NLANE, NLANE)] = 10.0 * b

            pltpu.sync_copy(bbuf, out_h.at[c].at[pl.ds(r0, RBLK)])
            pltpu.sync_copy(rb0, z_sh.at[pl.ds(r0, RBLK)])
            pltpu.sync_copy(rb0, s_sh.at[pl.ds(r0, RBLK)])

        plsc.subcore_barrier()

        def edge_pass():
            pltpu.sync_copy(srci_h.at[s].at[0], sb0)
            pltpu.sync_copy(dsti_h.at[s].at[0], db0)
            pltpu.async_copy(srci_h.at[s].at[1], sb1, isems)
            pltpu.async_copy(dsti_h.at[s].at[1], db1, isemd)
            pltpu.async_copy(z_sh.at[sb0.at[0]], rb0, gsem0)

            @pl.loop(0, n_iblk // 2)
            def _(ibp):
                for par in range(2):
                    ib = ibp * 2 + par
                    cs, cd = sbufs[par], dbufs[par]
                    ns, nd = sbufs[1 - par], dbufs[1 - par]
                    for ct in range(IB):
                        p = ct & 1
                        q = 1 - p
                        pltpu.make_async_copy(
                            z_sh.at[cs.at[ct]], rbufs[p], gsems[p]).wait()
                        if ct < IB - 1:
                            pltpu.async_copy(
                                z_sh.at[cs.at[ct + 1]], rbufs[q], gsems[q])
                            pltpu.sync_copy(
                                rbufs[p], s_sh.at[cd.at[ct]], add=True)
                        else:
                            pltpu.make_async_copy(
                                srci_h.at[s].at[ib + 1], ns, isems).wait()
                            pltpu.make_async_copy(
                                dsti_h.at[s].at[ib + 1], nd, isemd).wait()
                            pltpu.async_copy(
                                z_sh.at[ns.at[0]], rbufs[q], gsems[q])
                            pltpu.sync_copy(
                                rbufs[p], s_sh.at[cd.at[ct]], add=True)
                            pltpu.async_copy(
                                srci_h.at[s].at[ib + 2], cs, isems)
                            pltpu.async_copy(
                                dsti_h.at[s].at[ib + 2], cd, isemd)

            pltpu.make_async_copy(z_sh.at[sb0.at[0]], rb0, gsem0).wait()
            pltpu.make_async_copy(
                srci_h.at[s].at[n_iblk + 1], sb1, isems).wait()
            pltpu.make_async_copy(
                dsti_h.at[s].at[n_iblk + 1], db1, isemd).wait()

        @pl.loop(0, K_ITERS - 1)
        def _(k):
            edge_pass()
            plsc.subcore_barrier()

            for blk in range(N_RBLK):
                r0 = row0 + blk * RBLK
                pltpu.sync_copy(s_sh.at[pl.ds(r0, RBLK)], rb0)
                pltpu.sync_copy(out_h.at[c].at[pl.ds(r0, RBLK)], bbuf)

                @pl.loop(0, RBLK)
                def _(v):
                    vg = blk * RBLK + v
                    av = asl[vg, pl.ds(0, NLANE)]
                    for j in range(HALF // NLANE):
                        sj = rb0[v, pl.ds(j * NLANE, NLANE)]
                        rb0[v, pl.ds(j * NLANE, NLANE)] = (
                            av * sj + bbuf[v, pl.ds(j * NLANE, NLANE)]
                        )

                pltpu.sync_copy(rb0, z_sh.at[pl.ds(r0, RBLK)])
                pltpu.sync_copy(rb0, s_sh.at[pl.ds(r0, RBLK)])

            plsc.subcore_barrier()

        edge_pass()
        plsc.subcore_barrier()

        for blk in range(N_RBLK):
            r0 = row0 + blk * RBLK
            pltpu.sync_copy(s_sh.at[pl.ds(r0, RBLK)], rb0)
            pltpu.sync_copy(out_h.at[c].at[pl.ds(r0, RBLK)], bbuf)

            @pl.loop(0, RBLK)
            def _(v):
                vg = blk * RBLK + v
                av = asl[vg, pl.ds(0, NLANE)]
                rv = _fast_rsqrt(av * (1.0 / 0.9))
                for j in range(HALF // NLANE):
                    sj = rb0[v, pl.ds(j * NLANE, NLANE)]
                    zj = av * sj + bbuf[v, pl.ds(j * NLANE, NLANE)]
                    rb0[v, pl.ds(j * NLANE, NLANE)] = jnp.maximum(zj, 0.0) * rv

            pltpu.sync_copy(rb0, out_h.at[c].at[pl.ds(r0, RBLK)])

    return appnp


def kernel(x, edge_index):
    n, d = x.shape
    e = edge_index.shape[1]
    assert n == N_NODES and d == DIM

    unit = N_TILES * CH * IB
    n_iblk = -(-e // unit)
    if n_iblk % 2:
        n_iblk += 1
    epad = n_iblk * unit

    pad = N_NODES + (jnp.arange(epad - e, dtype=jnp.int32) % (NPAD - n))
    src = jnp.concatenate([edge_index[0], pad]).reshape(N_TILES, n_iblk, IB, CH)
    dst = jnp.concatenate([edge_index[1], pad]).reshape(N_TILES, n_iblk, IB, CH)
    src = jnp.pad(src, ((0, 0), (0, 2), (0, 0), (0, 0)),
                  constant_values=N_NODES)
    dst = jnp.pad(dst, ((0, 0), (0, 2), (0, 0), (0, 0)),
                  constant_values=N_NODES)

    xp = jnp.pad(x, ((0, NPAD - n), (0, 0)))
    xs = jnp.stack([xp[:, :HALF], xp[:, HALF:]])

    out = _make_appnp(n_iblk)(xs, src, dst)
    return jnp.concatenate([out[0, :n], out[1, :n]], axis=1)


# speedup vs baseline: 17.2485x; 1.0500x over previous
import functools

import jax
import jax.numpy as jnp
from jax import lax
from jax.experimental import pallas as pl
from jax.experimental.pallas import tpu as pltpu
from jax.experimental.pallas import tpu_sc as plsc

N_NODES = 10000
DIM = 128
HALF = DIM // 2
K_ITERS = 10

N_TILES = 16
NPAD = 10240
RPT = NPAD // N_TILES
RBLK = 128
N_RBLK = RPT // RBLK
CH = 128
IB = 8
NLANE = 16


def _fast_rsqrt(d):
    half = 0.5 * d
    i = lax.bitcast_convert_type(d, jnp.int32)
    y = lax.bitcast_convert_type(
        jnp.int32(0x5F3759DF) - lax.shift_right_arithmetic(i, 1), jnp.float32
    )
    for _ in range(3):
        y = y * (1.5 - half * y * y)
    return y


def _make_appnp(n_iblk):
    mesh = plsc.VectorSubcoreMesh(core_axis_name="c", subcore_axis_name="s")

    @functools.partial(
        pl.kernel,
        out_type=jax.ShapeDtypeStruct((2, NPAD, HALF), jnp.float32),
        mesh=mesh,
        compiler_params=pltpu.CompilerParams(use_tc_tiling_on_sc=False),
        scratch_types=[
            pltpu.VMEM_SHARED((NPAD, HALF), jnp.float32),
            pltpu.VMEM_SHARED((NPAD, HALF), jnp.float32),
            pltpu.VMEM((IB, CH), jnp.int32),
            pltpu.VMEM((IB, CH), jnp.int32),
            pltpu.VMEM((IB, CH), jnp.int32),
            pltpu.VMEM((IB, CH), jnp.int32),
            pltpu.VMEM((CH, HALF), jnp.float32),
            pltpu.VMEM((CH, HALF), jnp.float32),
            pltpu.VMEM((RBLK, HALF), jnp.float32),
            pltpu.VMEM((RPT, NLANE), jnp.float32),
            pltpu.VMEM((1, CH), jnp.int32),
            pltpu.SemaphoreType.DMA,
            pltpu.SemaphoreType.DMA,
            pltpu.SemaphoreType.DMA,
            pltpu.SemaphoreType.DMA,
            pltpu.SemaphoreType.DMA,
            pltpu.SemaphoreType.DMA,
        ],
    )
    def appnp(xs, srci_h, dsti_h, out_h, z_sh, s_sh, sb0, sb1, db0, db1,
              rb0, rb1, bbuf, asl, tidx, gsem0, gsem1, isems, isemd,
              ssem0, ssem1):
        ssem = ssem0
        c = lax.axis_index("c")
        s = lax.axis_index("s")
        row0 = s * RPT

        sbufs = (sb0, sb1)
        dbufs = (db0, db1)
        rbufs = (rb0, rb1)
        gsems = (gsem0, gsem1)
        ssems = (ssem0, ssem1)

        zero16 = jnp.zeros((NLANE,), jnp.float32)
        one16 = jnp.ones((NLANE,), jnp.float32)

        @pl.loop(0, CH)
        def _(i):
            for j in range(HALF // NLANE):
                rb0[i, pl.ds(j * NLANE, NLANE)] = zero16
                rb1[i, pl.ds(j * NLANE, NLANE)] = one16

        lane = jnp.arange(NLANE, dtype=jnp.int32)
        for g in range(CH // NLANE):
            tidx[0, pl.ds(g * NLANE, NLANE)] = N_NODES + (
                (lane + g * NLANE + s * 37) % (NPAD - N_NODES)
            )

        for blk in range(N_RBLK):
            pltpu.sync_copy(rb0, s_sh.at[pl.ds(row0 + blk * RBLK, RBLK)])
        plsc.subcore_barrier()

        pltpu.sync_copy(dsti_h.at[s].at[0], db0)
        pltpu.async_copy(dsti_h.at[s].at[1], db1, isemd)

        @pl.loop(0, n_iblk // 2)
        def _(ibp):
            for par in range(2):
                ib = ibp * 2 + par
                cd = dbufs[par]
                nd = dbufs[1 - par]
                for ct in range(IB):
                    pltpu.async_copy(rb1, s_sh.at[cd.at[ct]], ssem, add=True)
                for ct in range(IB):
                    pltpu.make_async_copy(
                        rb1, s_sh.at[cd.at[ct]], ssem).wait()
                pltpu.make_async_copy(dsti_h.at[s].at[ib + 1], nd, isemd).wait()
                pltpu.async_copy(dsti_h.at[s].at[ib + 2], cd, isemd)

        pltpu.make_async_copy(dsti_h.at[s].at[n_iblk + 1], db1, isemd).wait()
        plsc.subcore_barrier()

        for blk in range(N_RBLK):
            r0 = row0 + blk * RBLK
            pltpu.sync_copy(s_sh.at[pl.ds(r0, RBLK)], rb0)
            pltpu.sync_copy(xs.at[c].at[pl.ds(r0, RBLK)], bbuf)

            @pl.loop(0, RBLK)
            def _(v):
                vg = blk * RBLK + v
                deg = rb0[v, pl.ds(0, NLANE)] + 1.0
                dsv = _fast_rsqrt(deg)
                asl[vg, pl.ds(0, NLANE)] = 0.9 * dsv * dsv
                for j in range(HALF // NLANE):
                    b = 0.1 * dsv * bbuf[v, pl.ds(j * NLANE, NLANE)]
                    bbuf[v, pl.ds(j * NLANE, NLANE)] = b
                    rb0[v, pl.ds(j * NLANE, NLANE)] = 10.0 * b

            pltpu.sync_copy(bbuf, out_h.at[c].at[pl.ds(r0, RBLK)])
            pltpu.sync_copy(rb0, z_sh.at[pl.ds(r0, RBLK)])
            pltpu.sync_copy(rb0, s_sh.at[pl.ds(r0, RBLK)])

        plsc.subcore_barrier()

        def edge_pass():
            pltpu.sync_copy(srci_h.at[s].at[0], sb0)
            pltpu.sync_copy(dsti_h.at[s].at[0], db0)
            pltpu.async_copy(rb1, s_sh.at[tidx.at[0]], ssem1, add=True)
            pltpu.async_copy(z_sh.at[sb0.at[0]], rb0, gsem0)

            @pl.loop(0, n_iblk // 2)
            def _(ibp):
                for par in range(2):
                    ib = ibp * 2 + par
                    cs, cd = sbufs[par], dbufs[par]
                    ns, nd = sbufs[1 - par], dbufs[1 - par]
                    for ct in range(IB):
                        p = ct & 1
                        q = 1 - p
                        pltpu.make_async_copy(
                            z_sh.at[cs.at[ct]], rbufs[p], gsems[p]).wait()
                        pltpu.async_copy(
                            rbufs[p], s_sh.at[cd.at[ct]], ssems[p], add=True)
                        pltpu.make_async_copy(
                            rbufs[q], s_sh.at[cd.at[ct]], ssems[q]).wait()
                        if ct == 0:
                            pltpu.async_copy(srci_h.at[s].at[ib + 1], ns,
                                             isems)
                            pltpu.async_copy(dsti_h.at[s].at[ib + 1], nd,
                                             isemd)
                        if ct < IB - 1:
                            pltpu.async_copy(
                                z_sh.at[cs.at[ct + 1]], rbufs[q], gsems[q])
                        else:
                            pltpu.make_async_copy(
                                srci_h.at[s].at[ib + 1], ns, isems).wait()
                            pltpu.make_async_copy(
                                dsti_h.at[s].at[ib + 1], nd, isemd).wait()
                            pltpu.async_copy(
                                z_sh.at[ns.at[0]], rbufs[q], gsems[q])

            pltpu.make_async_copy(z_sh.at[sb0.at[0]], rb0, gsem0).wait()
            pltpu.make_async_copy(rb1, s_sh.at[tidx.at[0]], ssem1).wait()

        @pl.loop(0, K_ITERS - 1)
        def _(k):
            edge_pass()
            plsc.subcore_barrier()

            for blk in range(N_RBLK):
                r0 = row0 + blk * RBLK
                pltpu.sync_copy(s_sh.at[pl.ds(r0, RBLK)], rb0)
                pltpu.sync_copy(out_h.at[c].at[pl.ds(r0, RBLK)], bbuf)

                @pl.loop(0, RBLK)
                def _(v):
                    vg = blk * RBLK + v
                    av = asl[vg, pl.ds(0, NLANE)]
                    for j in range(HALF // NLANE):
                        sj = rb0[v, pl.ds(j * NLANE, NLANE)]
                        rb0[v, pl.ds(j * NLANE, NLANE)] = (
                            av * sj + bbuf[v, pl.ds(j * NLANE, NLANE)]
                        )

                pltpu.sync_copy(rb0, z_sh.at[pl.ds(r0, RBLK)])
                pltpu.sync_copy(rb0, s_sh.at[pl.ds(r0, RBLK)])

            plsc.subcore_barrier()

        edge_pass()
        plsc.subcore_barrier()

        for blk in range(N_RBLK):
            r0 = row0 + blk * RBLK
            pltpu.sync_copy(s_sh.at[pl.ds(r0, RBLK)], rb0)
            pltpu.sync_copy(out_h.at[c].at[pl.ds(r0, RBLK)], bbuf)

            @pl.loop(0, RBLK)
            def _(v):
                vg = blk * RBLK + v
                av = asl[vg, pl.ds(0, NLANE)]
                rv = _fast_rsqrt(av * (1.0 / 0.9))
                for j in range(HALF // NLANE):
                    sj = rb0[v, pl.ds(j * NLANE, NLANE)]
                    zj = av * sj + bbuf[v, pl.ds(j * NLANE, NLANE)]
                    rb0[v, pl.ds(j * NLANE, NLANE)] = jnp.maximum(zj, 0.0) * rv

            pltpu.sync_copy(rb0, out_h.at[c].at[pl.ds(r0, RBLK)])

    return appnp


def kernel(x, edge_index):
    n, d = x.shape
    e = edge_index.shape[1]
    assert n == N_NODES and d == DIM

    unit = N_TILES * CH * IB
    n_iblk = -(-e // unit)
    if n_iblk % 2:
        n_iblk += 1
    epad = n_iblk * unit

    pad = N_NODES + (jnp.arange(epad - e, dtype=jnp.int32) % (NPAD - n))
    src = jnp.concatenate([edge_index[0], pad]).reshape(N_TILES, n_iblk, IB, CH)
    dst = jnp.concatenate([edge_index[1], pad]).reshape(N_TILES, n_iblk, IB, CH)
    src = jnp.pad(src, ((0, 0), (0, 2), (0, 0), (0, 0)),
                  constant_values=N_NODES)
    dst = jnp.pad(dst, ((0, 0), (0, 2), (0, 0), (0, 0)),
                  constant_values=N_NODES)

    xp = jnp.pad(x, ((0, NPAD - n), (0, 0)))
    xs = jnp.stack([xp[:, :HALF], xp[:, HALF:]])

    out = _make_appnp(n_iblk)(xs, src, dst)
    return jnp.concatenate([out[0, :n], out[1, :n]], axis=1)


# speedup vs baseline: 18.0383x; 1.0458x over previous
import functools

import jax
import jax.numpy as jnp
from jax import lax
from jax.experimental import pallas as pl
from jax.experimental.pallas import tpu as pltpu
from jax.experimental.pallas import tpu_sc as plsc

N_NODES = 10000
DIM = 128
HALF = DIM // 2
K_ITERS = 10

N_TILES = 16
NPAD = 10240
RPT = NPAD // N_TILES
RBLK = 128
N_RBLK = RPT // RBLK
CH = 128
IB = 8
NLANE = 16


def _fast_rsqrt(d):
    half = 0.5 * d
    i = lax.bitcast_convert_type(d, jnp.int32)
    y = lax.bitcast_convert_type(
        jnp.int32(0x5F3759DF) - lax.shift_right_arithmetic(i, 1), jnp.float32
    )
    for _ in range(3):
        y = y * (1.5 - half * y * y)
    return y


def _make_appnp(n_iblk):
    mesh = plsc.VectorSubcoreMesh(core_axis_name="c", subcore_axis_name="s")

    @functools.partial(
        pl.kernel,
        out_type=jax.ShapeDtypeStruct((2, NPAD, HALF), jnp.float32),
        mesh=mesh,
        compiler_params=pltpu.CompilerParams(use_tc_tiling_on_sc=False),
        scratch_types=[
            pltpu.VMEM_SHARED((NPAD, HALF), jnp.float32),
            pltpu.VMEM_SHARED((NPAD, HALF), jnp.float32),
            pltpu.VMEM((IB, CH), jnp.int32),
            pltpu.VMEM((IB, CH), jnp.int32),
            pltpu.VMEM((IB, CH), jnp.int32),
            pltpu.VMEM((IB, CH), jnp.int32),
            pltpu.VMEM((CH, HALF), jnp.float32),
            pltpu.VMEM((CH, HALF), jnp.float32),
            pltpu.VMEM((RBLK, HALF), jnp.float32),
            pltpu.VMEM((RBLK, HALF), jnp.float32),
            pltpu.VMEM((RPT, NLANE), jnp.float32),
            pltpu.VMEM((1, CH), jnp.int32),
            pltpu.SemaphoreType.DMA,
            pltpu.SemaphoreType.DMA,
            pltpu.SemaphoreType.DMA,
            pltpu.SemaphoreType.DMA,
            pltpu.SemaphoreType.DMA,
            pltpu.SemaphoreType.DMA,
        ],
    )
    def appnp(xs, srci_h, dsti_h, out_h, z_sh, s_sh, sb0, sb1, db0, db1,
              rb0, rb1, bbuf, bbuf2, asl, tidx, gsem0, gsem1, isems,
              isemd, ssem0, ssem1):
        ssem = ssem0
        c = lax.axis_index("c")
        s = lax.axis_index("s")
        row0 = s * RPT

        sbufs = (sb0, sb1)
        dbufs = (db0, db1)
        rbufs = (rb0, rb1)
        gsems = (gsem0, gsem1)
        ssems = (ssem0, ssem1)

        zero16 = jnp.zeros((NLANE,), jnp.float32)
        one16 = jnp.ones((NLANE,), jnp.float32)

        @pl.loop(0, CH)
        def _(i):
            for j in range(HALF // NLANE):
                rb0[i, pl.ds(j * NLANE, NLANE)] = zero16
                rb1[i, pl.ds(j * NLANE, NLANE)] = one16

        lane = jnp.arange(NLANE, dtype=jnp.int32)
        for g in range(CH // NLANE):
            tidx[0, pl.ds(g * NLANE, NLANE)] = N_NODES + (
                (lane + g * NLANE + s * 37) % (NPAD - N_NODES)
            )

        for blk in range(N_RBLK):
            pltpu.sync_copy(rb0, s_sh.at[pl.ds(row0 + blk * RBLK, RBLK)])
        plsc.subcore_barrier()

        pltpu.sync_copy(dsti_h.at[s].at[0], db0)
        pltpu.async_copy(dsti_h.at[s].at[1], db1, isemd)

        @pl.loop(0, n_iblk // 2)
        def _(ibp):
            for par in range(2):
                ib = ibp * 2 + par
                cd = dbufs[par]
                nd = dbufs[1 - par]
                for ct in range(IB):
                    pltpu.async_copy(rb1, s_sh.at[cd.at[ct]], ssem, add=True)
                for ct in range(IB):
                    pltpu.make_async_copy(
                        rb1, s_sh.at[cd.at[ct]], ssem).wait()
                pltpu.make_async_copy(dsti_h.at[s].at[ib + 1], nd, isemd).wait()
                pltpu.async_copy(dsti_h.at[s].at[ib + 2], cd, isemd)

        pltpu.make_async_copy(dsti_h.at[s].at[n_iblk + 1], db1, isemd).wait()
        plsc.subcore_barrier()

        for blk in range(N_RBLK):
            r0 = row0 + blk * RBLK
            pltpu.sync_copy(s_sh.at[pl.ds(r0, RBLK)], rb0)
            pltpu.sync_copy(xs.at[c].at[pl.ds(r0, RBLK)], bbuf)

            @pl.loop(0, RBLK)
            def _(v):
                vg = blk * RBLK + v
                deg = rb0[v, pl.ds(0, NLANE)] + 1.0
                dsv = _fast_rsqrt(deg)
                asl[vg, pl.ds(0, NLANE)] = 0.9 * dsv * dsv
                for j in range(HALF // NLANE):
                    b = 0.1 * dsv * bbuf[v, pl.ds(j * NLANE, NLANE)]
                    bbuf[v, pl.ds(j * NLANE, NLANE)] = b
                    rb0[v, pl.ds(j * NLANE, NLANE)] = 10.0 * b

            pltpu.sync_copy(bbuf, out_h.at[c].at[pl.ds(r0, RBLK)])
            pltpu.sync_copy(rb0, z_sh.at[pl.ds(r0, RBLK)])
            pltpu.sync_copy(rb0, s_sh.at[pl.ds(r0, RBLK)])

        plsc.subcore_barrier()

        def edge_pass():
            pltpu.sync_copy(srci_h.at[s].at[0], sb0)
            pltpu.sync_copy(dsti_h.at[s].at[0], db0)
            pltpu.async_copy(rb1, s_sh.at[tidx.at[0]], ssem1, add=True)
            pltpu.async_copy(z_sh.at[sb0.at[0]], rb0, gsem0)

            @pl.loop(0, n_iblk // 2)
            def _(ibp):
                for par in range(2):
                    ib = ibp * 2 + par
                    cs, cd = sbufs[par], dbufs[par]
                    ns, nd = sbufs[1 - par], dbufs[1 - par]
                    for ct in range(IB):
                        p = ct & 1
                        q = 1 - p
                        pltpu.make_async_copy(
                            z_sh.at[cs.at[ct]], rbufs[p], gsems[p]).wait()
                        pltpu.async_copy(
                            rbufs[p], s_sh.at[cd.at[ct]], ssems[p], add=True)
                        pltpu.make_async_copy(
                            rbufs[q], s_sh.at[cd.at[ct]], ssems[q]).wait()
                        if ct == 0:
                            pltpu.async_copy(srci_h.at[s].at[ib + 1], ns,
                                             isems)
                            pltpu.async_copy(dsti_h.at[s].at[ib + 1], nd,
                                             isemd)
                        if ct < IB - 1:
                            pltpu.async_copy(
                                z_sh.at[cs.at[ct + 1]], rbufs[q], gsems[q])
                        else:
                            pltpu.make_async_copy(
                                srci_h.at[s].at[ib + 1], ns, isems).wait()
                            pltpu.make_async_copy(
                                dsti_h.at[s].at[ib + 1], nd, isemd).wait()
                            pltpu.async_copy(
                                z_sh.at[ns.at[0]], rbufs[q], gsems[q])

            pltpu.make_async_copy(z_sh.at[sb0.at[0]], rb0, gsem0).wait()
            pltpu.make_async_copy(rb1, s_sh.at[tidx.at[0]], ssem1).wait()

        bbufs = (bbuf, bbuf2)
        bsems = (isems, isemd)

        def update_phase(final):
            def stage(blk, p):
                r0 = row0 + blk * RBLK
                pltpu.async_copy(s_sh.at[pl.ds(r0, RBLK)], rbufs[p], gsems[p])
                pltpu.async_copy(out_h.at[c].at[pl.ds(r0, RBLK)], bbufs[p],
                                 bsems[p])

            def wait_wb(blk, p):
                r0 = row0 + blk * RBLK
                if final:
                    pltpu.make_async_copy(
                        rbufs[p], out_h.at[c].at[pl.ds(r0, RBLK)],
                        ssems[p]).wait()
                else:
                    pltpu.make_async_copy(
                        rbufs[p], z_sh.at[pl.ds(r0, RBLK)], ssems[p]).wait()
                    pltpu.make_async_copy(
                        rbufs[p], s_sh.at[pl.ds(r0, RBLK)], ssems[p]).wait()

            stage(0, 0)
            for blk in range(N_RBLK):
                p = blk & 1
                q = 1 - p
                r0 = row0 + blk * RBLK
                if blk + 1 < N_RBLK:
                    if blk >= 1:
                        wait_wb(blk - 1, q)
                    stage(blk + 1, q)
                pltpu.make_async_copy(
                    s_sh.at[pl.ds(r0, RBLK)], rbufs[p], gsems[p]).wait()
                pltpu.make_async_copy(
                    out_h.at[c].at[pl.ds(r0, RBLK)], bbufs[p],
                    bsems[p]).wait()
                rbb = rbufs[p]
                bbb = bbufs[p]

                @pl.loop(0, RBLK)
                def _(v):
                    vg = blk * RBLK + v
                    av = asl[vg, pl.ds(0, NLANE)]
                    if final:
                        rv = _fast_rsqrt(av * (1.0 / 0.9))
                    for j in range(HALF // NLANE):
                        sj = rbb[v, pl.ds(j * NLANE, NLANE)]
                        zj = av * sj + bbb[v, pl.ds(j * NLANE, NLANE)]
                        if final:
                            zj = jnp.maximum(zj, 0.0) * rv
                        rbb[v, pl.ds(j * NLANE, NLANE)] = zj

                if final:
                    pltpu.async_copy(rbb, out_h.at[c].at[pl.ds(r0, RBLK)],
                                     ssems[p])
                else:
                    pltpu.async_copy(rbb, z_sh.at[pl.ds(r0, RBLK)], ssems[p])
                    pltpu.async_copy(rbb, s_sh.at[pl.ds(r0, RBLK)], ssems[p])
            wait_wb(N_RBLK - 2, (N_RBLK - 2) & 1)
            wait_wb(N_RBLK - 1, (N_RBLK - 1) & 1)

        @pl.loop(0, K_ITERS - 1)
        def _(k):
            edge_pass()
            plsc.subcore_barrier()
            update_phase(final=False)
            plsc.subcore_barrier()

        edge_pass()
        plsc.subcore_barrier()
        update_phase(final=True)

    return appnp


def kernel(x, edge_index):
    n, d = x.shape
    e = edge_index.shape[1]
    assert n == N_NODES and d == DIM

    unit = N_TILES * CH * IB
    n_iblk = -(-e // unit)
    if n_iblk % 2:
        n_iblk += 1
    epad = n_iblk * unit

    pad = N_NODES + (jnp.arange(epad - e, dtype=jnp.int32) % (NPAD - n))
    src = jnp.concatenate([edge_index[0], pad]).reshape(N_TILES, n_iblk, IB, CH)
    dst = jnp.concatenate([edge_index[1], pad]).reshape(N_TILES, n_iblk, IB, CH)
    src = jnp.pad(src, ((0, 0), (0, 2), (0, 0), (0, 0)),
                  constant_values=N_NODES)
    dst = jnp.pad(dst, ((0, 0), (0, 2), (0, 0), (0, 0)),
                  constant_values=N_NODES)

    xp = jnp.pad(x, ((0, NPAD - n), (0, 0)))
    xs = jnp.stack([xp[:, :HALF], xp[:, HALF:]])

    out = _make_appnp(n_iblk)(xs, src, dst)
    return jnp.concatenate([out[0, :n], out[1, :n]], axis=1)
